# baseline (speedup 1.0000x reference)
import jax
import jax.numpy as jnp
from jax import lax
from jax.experimental import pallas as pl
from jax.experimental.pallas import tpu as pltpu
from jax.experimental.pallas import tpu_sc as plsc

_LANES = 16
_NUM_WORKERS = 32


def _make_kernel(H, W, F):
    assert H % _NUM_WORKERS == 0 or _NUM_WORKERS % H == 0
    rows_per_worker = max(1, H // _NUM_WORKERS)
    assert F % _LANES == 0
    nv = F // _LANES

    mesh = plsc.VectorSubcoreMesh(
        core_axis_name="c", subcore_axis_name="s", num_cores=2, num_subcores=16
    )

    def body(row_hbm, col_hbm, out_hbm, col_v, row_v, out_v):
        cid = lax.axis_index("c")
        sid = lax.axis_index("s")
        wid = sid * 2 + cid

        pltpu.sync_copy(col_hbm, col_v)
        for r in range(rows_per_worker):
            i = wid * rows_per_worker + r
            pltpu.sync_copy(row_hbm.at[i], row_v)
            for v in range(nv):
                reg = row_v[pl.ds(v * _LANES, _LANES)]
                for j in range(W):
                    out_v[j, pl.ds(F + v * _LANES, _LANES)] = reg
            for j in range(W):
                for v in range(nv):
                    out_v[j, pl.ds(v * _LANES, _LANES)] = col_v[
                        j, pl.ds(v * _LANES, _LANES)
                    ]
            pltpu.sync_copy(out_v, out_hbm.at[0, i])

    return pl.kernel(
        body,
        out_type=jax.ShapeDtypeStruct((1, H, W, 2 * F), jnp.float32),
        mesh=mesh,
        scratch_types=[
            pltpu.VMEM((W, F), jnp.float32),
            pltpu.VMEM((F,), jnp.float32),
            pltpu.VMEM((W, 2 * F), jnp.float32),
        ],
    )


def kernel(inputs, row_embed, col_embed):
    H = inputs.shape[1]
    W = inputs.shape[2]
    F = row_embed.shape[-1]
    return _make_kernel(H, W, F)(row_embed, col_embed)


# speedup vs baseline: 1.0094x; 1.0094x over previous
import jax
import jax.numpy as jnp
from jax import lax
from jax.experimental import pallas as pl
from jax.experimental.pallas import tpu as pltpu
from jax.experimental.pallas import tpu_sc as plsc

_LANES = 16
_NUM_WORKERS = 32


def _make_kernel(H, W, F):
    assert H == _NUM_WORKERS
    assert W % _LANES == 0 and F % _LANES == 0

    mesh = plsc.VectorSubcoreMesh(
        core_axis_name="c", subcore_axis_name="s", num_cores=2, num_subcores=16
    )

    def body(row_hbm, col_hbm, out_hbm, col_v, rb_v, idx_v, sem_in, sem_out):
        cid = lax.axis_index("c")
        sid = lax.axis_index("s")
        i = sid * 2 + cid

        splat = jnp.full((_LANES,), i, dtype=jnp.int32)
        for v in range(W // _LANES):
            idx_v[pl.ds(v * _LANES, _LANES)] = splat

        c_col = pltpu.async_copy(col_hbm, col_v, sem_in)
        c_row = pltpu.async_copy(row_hbm.at[idx_v], rb_v, sem_in)
        c_col.wait()
        c_row.wait()

        w_col = pltpu.async_copy(
            col_v, out_hbm.at[0, i, :, pl.ds(0, F)], sem_out
        )
        w_row = pltpu.async_copy(
            rb_v, out_hbm.at[0, i, :, pl.ds(F, F)], sem_out
        )
        w_col.wait()
        w_row.wait()

    return pl.kernel(
        body,
        out_type=jax.ShapeDtypeStruct((1, H, W, 2 * F), jnp.float32),
        mesh=mesh,
        scratch_types=[
            pltpu.VMEM((W, F), jnp.float32),
            pltpu.VMEM((W, F), jnp.float32),
            pltpu.VMEM((W,), jnp.int32),
            pltpu.SemaphoreType.DMA,
            pltpu.SemaphoreType.DMA,
        ],
    )


def kernel(inputs, row_embed, col_embed):
    H = inputs.shape[1]
    W = inputs.shape[2]
    F = row_embed.shape[-1]
    return _make_kernel(H, W, F)(row_embed, col_embed)


# speedup vs baseline: 1.0610x; 1.0511x over previous
import jax
import jax.numpy as jnp
from jax import lax
from jax.experimental import pallas as pl
from jax.experimental.pallas import tpu as pltpu
from jax.experimental.pallas import tpu_sc as plsc

_LANES = 16
_NUM_WORKERS = 32


def _make_kernel(H, W, F):
    assert H == _NUM_WORKERS
    assert W % _LANES == 0 and F % _LANES == 0

    mesh = plsc.VectorSubcoreMesh(
        core_axis_name="c", subcore_axis_name="s", num_cores=2, num_subcores=16
    )

    def body(row_hbm, col_hbm, out_hbm, out_v, row_v, sem_in):
        cid = lax.axis_index("c")
        sid = lax.axis_index("s")
        i = sid * 2 + cid

        c_col = pltpu.async_copy(col_hbm, out_v.at[:, pl.ds(0, F)], sem_in)
        c_row = pltpu.async_copy(row_hbm.at[i], row_v, sem_in)
        c_row.wait()
        for v in range(F // _LANES):
            reg = row_v[pl.ds(v * _LANES, _LANES)]
            for j in range(W):
                out_v[j, pl.ds(F + v * _LANES, _LANES)] = reg
        c_col.wait()
        pltpu.sync_copy(out_v, out_hbm.at[0, i])

    return pl.kernel(
        body,
        out_type=jax.ShapeDtypeStruct((1, H, W, 2 * F), jnp.float32),
        mesh=mesh,
        scratch_types=[
            pltpu.VMEM((W, 2 * F), jnp.float32),
            pltpu.VMEM((F,), jnp.float32),
            pltpu.SemaphoreType.DMA,
        ],
    )


def kernel(inputs, row_embed, col_embed):
    H = inputs.shape[1]
    W = inputs.shape[2]
    F = row_embed.shape[-1]
    return _make_kernel(H, W, F)(row_embed, col_embed)
